# baseline (speedup 1.0000x reference)
import functools

import jax
import jax.numpy as jnp
from jax import lax
from jax.experimental import pallas as pl
from jax.experimental.pallas import tpu as pltpu
from jax.experimental.pallas import tpu_sc as plsc

VOCAB = 100000
HIDDEN = 1024
TYPE_VOCAB = 2
MAX_POS = 8192
BATCH = 4
SEQ = 4096
LN_EPS = 1e-12

NTOK = BATCH * SEQ
NC = 2
NS = 16
NW = NC * NS

NSEG = 8
SEG_TOK = NTOK // NSEG
SEG_SEQ = SEQ // NSEG
ROWS_PER_W = SEG_TOK // NW
CHUNK = 32
N_CHUNKS = ROWS_PER_W // CHUNK

TOK_BLK = 256
BLKS_PER_SEQ = SEQ // TOK_BLK
SEG_BLKS = SEG_SEQ // TOK_BLK


def _sc_gather(table, idx):
    mesh = plsc.VectorSubcoreMesh(core_axis_name="c", subcore_axis_name="s")

    @functools.partial(
        pl.kernel,
        mesh=mesh,
        out_type=jax.ShapeDtypeStruct((SEG_TOK, HIDDEN), jnp.float32),
        scratch_types=[
            pltpu.VMEM((N_CHUNKS, CHUNK), jnp.int32),
            pltpu.VMEM((CHUNK, HIDDEN), jnp.float32),
            pltpu.VMEM((CHUNK, HIDDEN), jnp.float32),
            pltpu.SemaphoreType.DMA,
            pltpu.SemaphoreType.DMA,
            pltpu.SemaphoreType.DMA,
            pltpu.SemaphoreType.DMA,
        ],
    )
    def k(table_hbm, idx_hbm, out_hbm, idx_v, rows0, rows1, gsem0, gsem1,
          osem0, osem1):
        wid = lax.axis_index("s") * NC + lax.axis_index("c")
        base = wid * ROWS_PER_W
        pltpu.sync_copy(idx_hbm.at[wid], idx_v)

        rows = (rows0, rows1)
        gsem = (gsem0, gsem1)
        osem = (osem0, osem1)

        def out_slice(j):
            return out_hbm.at[pl.ds(base + j * CHUNK, CHUNK)]

        pltpu.async_copy(table_hbm.at[idx_v.at[0]], rows[0], gsem[0])
        for j in range(N_CHUNKS):
            b = j % 2
            if j + 1 < N_CHUNKS:
                nb = (j + 1) % 2
                if j >= 1:
                    pltpu.make_async_copy(rows[nb], out_slice(j - 1),
                                          osem[nb]).wait()
                pltpu.async_copy(table_hbm.at[idx_v.at[j + 1]], rows[nb],
                                 gsem[nb])
            pltpu.make_async_copy(table_hbm.at[idx_v.at[j]], rows[b],
                                  gsem[b]).wait()
            pltpu.async_copy(rows[b], out_slice(j), osem[b])
        for j in (N_CHUNKS - 2, N_CHUNKS - 1):
            pltpu.make_async_copy(rows[j % 2], out_slice(j),
                                  osem[j % 2]).wait()

    return k(table, idx)


def _ln_math(g_ref, tt_ref, ttab_ref, pos_ref, gamma_ref, beta_ref, out_ref):
    x = g_ref[...]
    tt = tt_ref[0, 0, :]
    row0 = ttab_ref[0, :]
    row1 = ttab_ref[1, :]
    ttf = tt.astype(jnp.float32)[:, None]
    x = x + row0[None, :] + ttf * (row1 - row0)[None, :]
    x = x + pos_ref[...]
    mean = jnp.mean(x, axis=1, keepdims=True)
    xc = x - mean
    var = jnp.mean(xc * xc, axis=1, keepdims=True)
    normed = xc * lax.rsqrt(var + LN_EPS)
    out_ref[...] = normed * gamma_ref[0, :][None, :] + beta_ref[0, :][None, :]


def _ln_body_first(g_ref, tt_ref, ttab_ref, pos_ref, gamma_ref, beta_ref,
                   out_ref):
    _ln_math(g_ref, tt_ref, ttab_ref, pos_ref, gamma_ref, beta_ref, out_ref)


def _ln_body_chained(o_prev_ref, g_ref, tt_ref, ttab_ref, pos_ref, gamma_ref,
                     beta_ref, out_ref):
    del o_prev_ref
    _ln_math(g_ref, tt_ref, ttab_ref, pos_ref, gamma_ref, beta_ref, out_ref)


def _tc_add_ln_seg(seg, o_prev, gathered, tt_seg, token_type_embeddings,
                   position_embeddings, gamma2d, beta2d):
    tt3 = tt_seg.reshape(SEG_TOK // TOK_BLK, 1, TOK_BLK)
    data_specs = [
        pl.BlockSpec((TOK_BLK, HIDDEN), lambda i, j: (j * SEG_BLKS + i, 0)),
        pl.BlockSpec((1, 1, TOK_BLK), lambda i, j: (j * SEG_BLKS + i, 0, 0)),
        pl.BlockSpec((TYPE_VOCAB, HIDDEN), lambda i, j: (0, 0)),
        pl.BlockSpec((TOK_BLK, HIDDEN), lambda i, j: (seg * SEG_BLKS + i, 0)),
        pl.BlockSpec((1, HIDDEN), lambda i, j: (0, 0)),
        pl.BlockSpec((1, HIDDEN), lambda i, j: (0, 0)),
    ]
    out_spec = pl.BlockSpec(
        (TOK_BLK, HIDDEN),
        lambda i, j: (j * BLKS_PER_SEQ + seg * SEG_BLKS + i, 0))
    out_shape = jax.ShapeDtypeStruct((NTOK, HIDDEN), jnp.float32)
    args = (gathered, tt3, token_type_embeddings, position_embeddings,
            gamma2d, beta2d)
    if o_prev is None:
        return pl.pallas_call(
            _ln_body_first,
            grid=(SEG_BLKS, BATCH),
            in_specs=data_specs,
            out_specs=out_spec,
            out_shape=out_shape,
        )(*args)
    return pl.pallas_call(
        _ln_body_chained,
        grid=(SEG_BLKS, BATCH),
        in_specs=[pl.BlockSpec(memory_space=pltpu.MemorySpace.HBM)] + data_specs,
        out_specs=out_spec,
        out_shape=out_shape,
        input_output_aliases={0: 0},
    )(o_prev, *args)


@jax.jit
def kernel(input_ids, token_type_ids, word_embeddings, position_embeddings,
           token_type_embeddings, ln_gamma, ln_beta):
    ids = input_ids.astype(jnp.int32)
    tts = token_type_ids.astype(jnp.int32)
    gamma2d = ln_gamma.reshape(1, HIDDEN)
    beta2d = ln_beta.reshape(1, HIDDEN)

    out = None
    for seg in range(NSEG):
        s0 = seg * SEG_SEQ
        idx_seg = ids[:, s0:s0 + SEG_SEQ].reshape(NW, N_CHUNKS, CHUNK)
        tt_seg = tts[:, s0:s0 + SEG_SEQ].reshape(-1)
        gathered = _sc_gather(word_embeddings, idx_seg)
        out = _tc_add_ln_seg(seg, out, gathered, tt_seg,
                             token_type_embeddings, position_embeddings,
                             gamma2d, beta2d)
    return out.reshape(BATCH, SEQ, HIDDEN)


# speedup vs baseline: 1.0281x; 1.0281x over previous
import functools

import jax
import jax.numpy as jnp
from jax import lax
from jax.experimental import pallas as pl
from jax.experimental.pallas import tpu as pltpu
from jax.experimental.pallas import tpu_sc as plsc

VOCAB = 100000
HIDDEN = 1024
TYPE_VOCAB = 2
MAX_POS = 8192
BATCH = 4
SEQ = 4096
LN_EPS = 1e-12

NTOK = BATCH * SEQ
NC = 2
NS = 16
NW = NC * NS

NSEG = 4
SEG_TOK = NTOK // NSEG
SEG_SEQ = SEQ // NSEG
ROWS_PER_W = SEG_TOK // NW
CHUNK = 16
N_CHUNKS = ROWS_PER_W // CHUNK
NBUF = 4

TOK_BLK = 256
BLKS_PER_SEQ = SEQ // TOK_BLK
SEG_BLKS = SEG_SEQ // TOK_BLK


def _sc_gather(table, idx):
    mesh = plsc.VectorSubcoreMesh(core_axis_name="c", subcore_axis_name="s")

    @functools.partial(
        pl.kernel,
        mesh=mesh,
        out_type=jax.ShapeDtypeStruct((SEG_TOK, HIDDEN), jnp.float32),
        scratch_types=(
            [pltpu.VMEM((N_CHUNKS, CHUNK), jnp.int32)]
            + [pltpu.VMEM((CHUNK, HIDDEN), jnp.float32)] * NBUF
            + [pltpu.SemaphoreType.DMA] * (2 * NBUF)
        ),
    )
    def k(table_hbm, idx_hbm, out_hbm, idx_v, *bufs_and_sems):
        rows = bufs_and_sems[:NBUF]
        gsem = bufs_and_sems[NBUF:2 * NBUF]
        osem = bufs_and_sems[2 * NBUF:3 * NBUF]
        wid = lax.axis_index("s") * NC + lax.axis_index("c")
        base = wid * ROWS_PER_W
        pltpu.sync_copy(idx_hbm.at[wid], idx_v)

        def out_slice(j):
            return out_hbm.at[pl.ds(base + j * CHUNK, CHUNK)]

        def start_gather(j):
            pltpu.async_copy(table_hbm.at[idx_v.at[j]], rows[j % NBUF],
                             gsem[j % NBUF])

        for j in range(min(2, N_CHUNKS)):
            start_gather(j)
        for j in range(N_CHUNKS):
            b = j % NBUF
            if j + 2 < N_CHUNKS:
                nb = (j + 2) % NBUF
                if j >= 2:
                    pltpu.make_async_copy(rows[nb], out_slice(j - 2),
                                          osem[nb]).wait()
                start_gather(j + 2)
            pltpu.make_async_copy(table_hbm.at[idx_v.at[j]], rows[b],
                                  gsem[b]).wait()
            pltpu.async_copy(rows[b], out_slice(j), osem[b])
        for j in range(max(0, N_CHUNKS - NBUF), N_CHUNKS):
            pltpu.make_async_copy(rows[j % NBUF], out_slice(j),
                                  osem[j % NBUF]).wait()

    return k(table, idx)


def _ln_math(g_ref, tt_ref, ttab_ref, pos_ref, gamma_ref, beta_ref, out_ref):
    x = g_ref[...]
    tt = tt_ref[0, 0, :]
    row0 = ttab_ref[0, :]
    row1 = ttab_ref[1, :]
    ttf = tt.astype(jnp.float32)[:, None]
    x = x + row0[None, :] + ttf * (row1 - row0)[None, :]
    x = x + pos_ref[...]
    mean = jnp.mean(x, axis=1, keepdims=True)
    xc = x - mean
    var = jnp.mean(xc * xc, axis=1, keepdims=True)
    normed = xc * lax.rsqrt(var + LN_EPS)
    out_ref[...] = normed * gamma_ref[0, :][None, :] + beta_ref[0, :][None, :]


def _ln_body_first(g_ref, tt_ref, ttab_ref, pos_ref, gamma_ref, beta_ref,
                   out_ref):
    _ln_math(g_ref, tt_ref, ttab_ref, pos_ref, gamma_ref, beta_ref, out_ref)


def _ln_body_chained(o_prev_ref, g_ref, tt_ref, ttab_ref, pos_ref, gamma_ref,
                     beta_ref, out_ref):
    del o_prev_ref
    _ln_math(g_ref, tt_ref, ttab_ref, pos_ref, gamma_ref, beta_ref, out_ref)


def _tc_add_ln_seg(seg, o_prev, gathered, tt_seg, token_type_embeddings,
                   position_embeddings, gamma2d, beta2d):
    tt3 = tt_seg.reshape(SEG_TOK // TOK_BLK, 1, TOK_BLK)
    data_specs = [
        pl.BlockSpec((TOK_BLK, HIDDEN), lambda i, j: (j * SEG_BLKS + i, 0)),
        pl.BlockSpec((1, 1, TOK_BLK), lambda i, j: (j * SEG_BLKS + i, 0, 0)),
        pl.BlockSpec((TYPE_VOCAB, HIDDEN), lambda i, j: (0, 0)),
        pl.BlockSpec((TOK_BLK, HIDDEN), lambda i, j: (seg * SEG_BLKS + i, 0)),
        pl.BlockSpec((1, HIDDEN), lambda i, j: (0, 0)),
        pl.BlockSpec((1, HIDDEN), lambda i, j: (0, 0)),
    ]
    out_spec = pl.BlockSpec(
        (TOK_BLK, HIDDEN),
        lambda i, j: (j * BLKS_PER_SEQ + seg * SEG_BLKS + i, 0))
    out_shape = jax.ShapeDtypeStruct((NTOK, HIDDEN), jnp.float32)
    args = (gathered, tt3, token_type_embeddings, position_embeddings,
            gamma2d, beta2d)
    if o_prev is None:
        return pl.pallas_call(
            _ln_body_first,
            grid=(SEG_BLKS, BATCH),
            in_specs=data_specs,
            out_specs=out_spec,
            out_shape=out_shape,
        )(*args)
    return pl.pallas_call(
        _ln_body_chained,
        grid=(SEG_BLKS, BATCH),
        in_specs=[pl.BlockSpec(memory_space=pltpu.MemorySpace.HBM)] + data_specs,
        out_specs=out_spec,
        out_shape=out_shape,
        input_output_aliases={0: 0},
    )(o_prev, *args)


@jax.jit
def kernel(input_ids, token_type_ids, word_embeddings, position_embeddings,
           token_type_embeddings, ln_gamma, ln_beta):
    ids = input_ids.astype(jnp.int32)
    tts = token_type_ids.astype(jnp.int32)
    gamma2d = ln_gamma.reshape(1, HIDDEN)
    beta2d = ln_beta.reshape(1, HIDDEN)

    out = None
    for seg in range(NSEG):
        s0 = seg * SEG_SEQ
        idx_seg = ids[:, s0:s0 + SEG_SEQ].reshape(NW, N_CHUNKS, CHUNK)
        tt_seg = tts[:, s0:s0 + SEG_SEQ].reshape(-1)
        gathered = _sc_gather(word_embeddings, idx_seg)
        out = _tc_add_ln_seg(seg, out, gathered, tt_seg,
                             token_type_embeddings, position_embeddings,
                             gamma2d, beta2d)
    return out.reshape(BATCH, SEQ, HIDDEN)


# speedup vs baseline: 1.1317x; 1.1007x over previous
import functools

import jax
import jax.numpy as jnp
from jax import lax
from jax.experimental import pallas as pl
from jax.experimental.pallas import tpu as pltpu
from jax.experimental.pallas import tpu_sc as plsc

VOCAB = 100000
HIDDEN = 1024
TYPE_VOCAB = 2
MAX_POS = 8192
BATCH = 4
SEQ = 4096
LN_EPS = 1e-12

NTOK = BATCH * SEQ
NC = 2
NS = 16
NW = NC * NS

NSEG = 4
SEG_TOK = NTOK // NSEG
SEG_SEQ = SEQ // NSEG
ROWS_PER_W = SEG_TOK // NW
CHUNK = 16
N_CHUNKS = ROWS_PER_W // CHUNK
NBUF = 4

TOK_BLK = 512
BLKS_PER_SEQ = SEQ // TOK_BLK
SEG_BLKS = SEG_SEQ // TOK_BLK


def _sc_gather(table, idx):
    mesh = plsc.VectorSubcoreMesh(core_axis_name="c", subcore_axis_name="s")

    @functools.partial(
        pl.kernel,
        mesh=mesh,
        out_type=jax.ShapeDtypeStruct((SEG_TOK, HIDDEN), jnp.float32),
        scratch_types=(
            [pltpu.VMEM((N_CHUNKS, CHUNK), jnp.int32)]
            + [pltpu.VMEM((CHUNK, HIDDEN), jnp.float32)] * NBUF
            + [pltpu.SemaphoreType.DMA] * (2 * NBUF)
        ),
    )
    def k(table_hbm, idx_hbm, out_hbm, idx_v, *bufs_and_sems):
        rows = bufs_and_sems[:NBUF]
        gsem = bufs_and_sems[NBUF:2 * NBUF]
        osem = bufs_and_sems[2 * NBUF:3 * NBUF]
        wid = lax.axis_index("s") * NC + lax.axis_index("c")
        base = wid * ROWS_PER_W
        pltpu.sync_copy(idx_hbm.at[wid], idx_v)

        def out_slice(j):
            return out_hbm.at[pl.ds(base + j * CHUNK, CHUNK)]

        def start_gather(j):
            pltpu.async_copy(table_hbm.at[idx_v.at[j]], rows[j % NBUF],
                             gsem[j % NBUF])

        for j in range(min(2, N_CHUNKS)):
            start_gather(j)
        for j in range(N_CHUNKS):
            b = j % NBUF
            if j + 2 < N_CHUNKS:
                nb = (j + 2) % NBUF
                if j >= 2:
                    pltpu.make_async_copy(rows[nb], out_slice(j - 2),
                                          osem[nb]).wait()
                start_gather(j + 2)
            pltpu.make_async_copy(table_hbm.at[idx_v.at[j]], rows[b],
                                  gsem[b]).wait()
            pltpu.async_copy(rows[b], out_slice(j), osem[b])
        for j in range(max(0, N_CHUNKS - NBUF), N_CHUNKS):
            pltpu.make_async_copy(rows[j % NBUF], out_slice(j),
                                  osem[j % NBUF]).wait()

    return k(table, idx)


def _ln_math(g_ref, tt_ref, ttab_ref, pos_ref, gamma_ref, beta_ref, out_ref):
    x = g_ref[...]
    tt = tt_ref[0, 0, :]
    row0 = ttab_ref[0, :]
    row1 = ttab_ref[1, :]
    ttf = tt.astype(jnp.float32)[:, None]
    x = x + row0[None, :] + ttf * (row1 - row0)[None, :]
    x = x + pos_ref[...]
    mean = jnp.mean(x, axis=1, keepdims=True)
    xc = x - mean
    var = jnp.mean(xc * xc, axis=1, keepdims=True)
    normed = xc * lax.rsqrt(var + LN_EPS)
    out_ref[...] = normed * gamma_ref[0, :][None, :] + beta_ref[0, :][None, :]


def _ln_body_first(g_ref, tt_ref, ttab_ref, pos_ref, gamma_ref, beta_ref,
                   out_ref):
    _ln_math(g_ref, tt_ref, ttab_ref, pos_ref, gamma_ref, beta_ref, out_ref)


def _ln_body_chained(o_prev_ref, g_ref, tt_ref, ttab_ref, pos_ref, gamma_ref,
                     beta_ref, out_ref):
    del o_prev_ref
    _ln_math(g_ref, tt_ref, ttab_ref, pos_ref, gamma_ref, beta_ref, out_ref)


def _tc_add_ln_seg(seg, o_prev, gathered, tt_seg, token_type_embeddings,
                   position_embeddings, gamma2d, beta2d):
    tt3 = tt_seg.reshape(SEG_TOK // TOK_BLK, 1, TOK_BLK)
    data_specs = [
        pl.BlockSpec((TOK_BLK, HIDDEN), lambda i, j: (j * SEG_BLKS + i, 0)),
        pl.BlockSpec((1, 1, TOK_BLK), lambda i, j: (j * SEG_BLKS + i, 0, 0)),
        pl.BlockSpec((TYPE_VOCAB, HIDDEN), lambda i, j: (0, 0)),
        pl.BlockSpec((TOK_BLK, HIDDEN), lambda i, j: (seg * SEG_BLKS + i, 0)),
        pl.BlockSpec((1, HIDDEN), lambda i, j: (0, 0)),
        pl.BlockSpec((1, HIDDEN), lambda i, j: (0, 0)),
    ]
    out_spec = pl.BlockSpec(
        (TOK_BLK, HIDDEN),
        lambda i, j: (j * BLKS_PER_SEQ + seg * SEG_BLKS + i, 0))
    out_shape = jax.ShapeDtypeStruct((NTOK, HIDDEN), jnp.float32)
    args = (gathered, tt3, token_type_embeddings, position_embeddings,
            gamma2d, beta2d)
    if o_prev is None:
        return pl.pallas_call(
            _ln_body_first,
            grid=(SEG_BLKS, BATCH),
            in_specs=data_specs,
            out_specs=out_spec,
            out_shape=out_shape,
        )(*args)
    return pl.pallas_call(
        _ln_body_chained,
        grid=(SEG_BLKS, BATCH),
        in_specs=[pl.BlockSpec(memory_space=pltpu.MemorySpace.HBM)] + data_specs,
        out_specs=out_spec,
        out_shape=out_shape,
        input_output_aliases={0: 0},
    )(o_prev, *args)


@jax.jit
def kernel(input_ids, token_type_ids, word_embeddings, position_embeddings,
           token_type_embeddings, ln_gamma, ln_beta):
    ids = input_ids.astype(jnp.int32)
    tts = token_type_ids.astype(jnp.int32)
    gamma2d = ln_gamma.reshape(1, HIDDEN)
    beta2d = ln_beta.reshape(1, HIDDEN)

    out = None
    for seg in range(NSEG):
        s0 = seg * SEG_SEQ
        idx_seg = ids[:, s0:s0 + SEG_SEQ].reshape(NW, N_CHUNKS, CHUNK)
        tt_seg = tts[:, s0:s0 + SEG_SEQ].reshape(-1)
        gathered = _sc_gather(word_embeddings, idx_seg)
        out = _tc_add_ln_seg(seg, out, gathered, tt_seg,
                             token_type_embeddings, position_embeddings,
                             gamma2d, beta2d)
    return out.reshape(BATCH, SEQ, HIDDEN)


# speedup vs baseline: 1.1639x; 1.0285x over previous
import functools

import jax
import jax.numpy as jnp
from jax import lax
from jax.experimental import pallas as pl
from jax.experimental.pallas import tpu as pltpu
from jax.experimental.pallas import tpu_sc as plsc

VOCAB = 100000
HIDDEN = 1024
TYPE_VOCAB = 2
MAX_POS = 8192
BATCH = 4
SEQ = 4096
LN_EPS = 1e-12

NTOK = BATCH * SEQ
NC = 2
NS = 16
NW = NC * NS

NSEG = 4
SEG_TOK = NTOK // NSEG
SEG_SEQ = SEQ // NSEG
ROWS_PER_W = SEG_TOK // NW
CHUNK = 16
N_CHUNKS = ROWS_PER_W // CHUNK
NBUF = 4

TOK_BLK = 1024
BLKS_PER_SEQ = SEQ // TOK_BLK
SEG_BLKS = SEG_SEQ // TOK_BLK


def _sc_gather(table, idx):
    mesh = plsc.VectorSubcoreMesh(core_axis_name="c", subcore_axis_name="s")

    @functools.partial(
        pl.kernel,
        mesh=mesh,
        out_type=jax.ShapeDtypeStruct((SEG_TOK, HIDDEN), jnp.float32),
        scratch_types=(
            [pltpu.VMEM((N_CHUNKS, CHUNK), jnp.int32)]
            + [pltpu.VMEM((CHUNK, HIDDEN), jnp.float32)] * NBUF
            + [pltpu.SemaphoreType.DMA] * (2 * NBUF)
        ),
    )
    def k(table_hbm, idx_hbm, out_hbm, idx_v, *bufs_and_sems):
        rows = bufs_and_sems[:NBUF]
        gsem = bufs_and_sems[NBUF:2 * NBUF]
        osem = bufs_and_sems[2 * NBUF:3 * NBUF]
        wid = lax.axis_index("s") * NC + lax.axis_index("c")
        base = wid * ROWS_PER_W
        pltpu.sync_copy(idx_hbm.at[wid], idx_v)

        def out_slice(j):
            return out_hbm.at[pl.ds(base + j * CHUNK, CHUNK)]

        def start_gather(j):
            pltpu.async_copy(table_hbm.at[idx_v.at[j]], rows[j % NBUF],
                             gsem[j % NBUF])

        for j in range(min(2, N_CHUNKS)):
            start_gather(j)
        for j in range(N_CHUNKS):
            b = j % NBUF
            if j + 2 < N_CHUNKS:
                nb = (j + 2) % NBUF
                if j >= 2:
                    pltpu.make_async_copy(rows[nb], out_slice(j - 2),
                                          osem[nb]).wait()
                start_gather(j + 2)
            pltpu.make_async_copy(table_hbm.at[idx_v.at[j]], rows[b],
                                  gsem[b]).wait()
            pltpu.async_copy(rows[b], out_slice(j), osem[b])
        for j in range(max(0, N_CHUNKS - NBUF), N_CHUNKS):
            pltpu.make_async_copy(rows[j % NBUF], out_slice(j),
                                  osem[j % NBUF]).wait()

    return k(table, idx)


def _ln_math(g_ref, tt_ref, ttab_ref, pos_ref, gamma_ref, beta_ref, out_ref):
    x = g_ref[...]
    tt = tt_ref[0, 0, :]
    row0 = ttab_ref[0, :]
    row1 = ttab_ref[1, :]
    ttf = tt.astype(jnp.float32)[:, None]
    x = x + row0[None, :] + ttf * (row1 - row0)[None, :]
    x = x + pos_ref[...]
    mean = jnp.mean(x, axis=1, keepdims=True)
    xc = x - mean
    var = jnp.mean(xc * xc, axis=1, keepdims=True)
    normed = xc * lax.rsqrt(var + LN_EPS)
    out_ref[...] = normed * gamma_ref[0, :][None, :] + beta_ref[0, :][None, :]


def _ln_body_first(g_ref, tt_ref, ttab_ref, pos_ref, gamma_ref, beta_ref,
                   out_ref):
    _ln_math(g_ref, tt_ref, ttab_ref, pos_ref, gamma_ref, beta_ref, out_ref)


def _ln_body_chained(o_prev_ref, g_ref, tt_ref, ttab_ref, pos_ref, gamma_ref,
                     beta_ref, out_ref):
    del o_prev_ref
    _ln_math(g_ref, tt_ref, ttab_ref, pos_ref, gamma_ref, beta_ref, out_ref)


def _tc_add_ln_seg(seg, o_prev, gathered, tt_seg, token_type_embeddings,
                   position_embeddings, gamma2d, beta2d):
    tt3 = tt_seg.reshape(SEG_TOK // TOK_BLK, 1, TOK_BLK)
    data_specs = [
        pl.BlockSpec((TOK_BLK, HIDDEN), lambda i, j: (j * SEG_BLKS + i, 0)),
        pl.BlockSpec((1, 1, TOK_BLK), lambda i, j: (j * SEG_BLKS + i, 0, 0)),
        pl.BlockSpec((TYPE_VOCAB, HIDDEN), lambda i, j: (0, 0)),
        pl.BlockSpec((TOK_BLK, HIDDEN), lambda i, j: (seg * SEG_BLKS + i, 0)),
        pl.BlockSpec((1, HIDDEN), lambda i, j: (0, 0)),
        pl.BlockSpec((1, HIDDEN), lambda i, j: (0, 0)),
    ]
    out_spec = pl.BlockSpec(
        (TOK_BLK, HIDDEN),
        lambda i, j: (j * BLKS_PER_SEQ + seg * SEG_BLKS + i, 0))
    out_shape = jax.ShapeDtypeStruct((NTOK, HIDDEN), jnp.float32)
    args = (gathered, tt3, token_type_embeddings, position_embeddings,
            gamma2d, beta2d)
    if o_prev is None:
        return pl.pallas_call(
            _ln_body_first,
            grid=(SEG_BLKS, BATCH),
            in_specs=data_specs,
            out_specs=out_spec,
            out_shape=out_shape,
        )(*args)
    return pl.pallas_call(
        _ln_body_chained,
        grid=(SEG_BLKS, BATCH),
        in_specs=[pl.BlockSpec(memory_space=pltpu.MemorySpace.HBM)] + data_specs,
        out_specs=out_spec,
        out_shape=out_shape,
        input_output_aliases={0: 0},
    )(o_prev, *args)


@jax.jit
def kernel(input_ids, token_type_ids, word_embeddings, position_embeddings,
           token_type_embeddings, ln_gamma, ln_beta):
    ids = input_ids.astype(jnp.int32)
    tts = token_type_ids.astype(jnp.int32)
    gamma2d = ln_gamma.reshape(1, HIDDEN)
    beta2d = ln_beta.reshape(1, HIDDEN)

    out = None
    for seg in range(NSEG):
        s0 = seg * SEG_SEQ
        idx_seg = ids[:, s0:s0 + SEG_SEQ].reshape(NW, N_CHUNKS, CHUNK)
        tt_seg = tts[:, s0:s0 + SEG_SEQ].reshape(-1)
        gathered = _sc_gather(word_embeddings, idx_seg)
        out = _tc_add_ln_seg(seg, out, gathered, tt_seg,
                             token_type_embeddings, position_embeddings,
                             gamma2d, beta2d)
    return out.reshape(BATCH, SEQ, HIDDEN)
